# baseline (speedup 1.0000x reference)
import functools

import jax
import jax.numpy as jnp
from jax import lax
from jax.experimental import pallas as pl
from jax.experimental.pallas import tpu as pltpu
from jax.experimental.pallas import tpu_sc as plsc

_BATCH = 16384
_HIST = 200
_DIM = 64
_TOTAL = _BATCH * _HIST
_NW = 32
_PER_W = _TOTAL // _NW
_CHUNK = 512
_N_CHUNKS = _PER_W // _CHUNK

_mesh = plsc.VectorSubcoreMesh(core_axis_name="c", subcore_axis_name="s")


@functools.partial(
    pl.kernel,
    out_type=jax.ShapeDtypeStruct((_TOTAL, _DIM), jnp.float32),
    mesh=_mesh,
    scratch_types=[
        pltpu.VMEM((_CHUNK,), jnp.int32),
        pltpu.VMEM((_CHUNK, _DIM), jnp.float32),
        pltpu.SemaphoreType.DMA,
    ],
    compiler_params=pltpu.CompilerParams(use_tc_tiling_on_sc=False),
)
def _gather_all(idx_hbm, table_hbm, out_hbm, idx_v, rows_v, sem):
    wid = lax.axis_index("s") * 2 + lax.axis_index("c")
    base = wid * _PER_W

    def body(g, carry):
        off = base + g * _CHUNK
        pltpu.sync_copy(idx_hbm.at[pl.ds(off, _CHUNK)], idx_v)
        pltpu.async_copy(table_hbm.at[idx_v], rows_v, sem).wait()
        pltpu.sync_copy(rows_v, out_hbm.at[pl.ds(off, _CHUNK)])
        return carry

    lax.fori_loop(0, _N_CHUNKS, body, 0)


def kernel(node_ids, embedding_weight):
    idx = node_ids.reshape(_TOTAL).astype(jnp.int32)
    out = _gather_all(idx, embedding_weight)
    return out.reshape(_BATCH, _HIST, _DIM)


# speedup vs baseline: 1.0755x; 1.0755x over previous
import functools

import jax
import jax.numpy as jnp
from jax import lax
from jax.experimental import pallas as pl
from jax.experimental.pallas import tpu as pltpu
from jax.experimental.pallas import tpu_sc as plsc

_BATCH = 16384
_HIST = 200
_DIM = 64
_TOTAL = _BATCH * _HIST
_NW = 32
_PER_W = _TOTAL // _NW
_CHUNK = 800
_N_CHUNKS = _PER_W // _CHUNK

_mesh = plsc.VectorSubcoreMesh(core_axis_name="c", subcore_axis_name="s")


@functools.partial(
    pl.kernel,
    out_type=jax.ShapeDtypeStruct((_TOTAL, _DIM), jnp.float32),
    mesh=_mesh,
    scratch_types=[
        pltpu.VMEM((2, _CHUNK), jnp.int32),
        pltpu.VMEM((2, _CHUNK, _DIM), jnp.float32),
        pltpu.SemaphoreType.DMA,
        pltpu.SemaphoreType.DMA,
        pltpu.SemaphoreType.DMA,
        pltpu.SemaphoreType.DMA,
        pltpu.SemaphoreType.DMA,
        pltpu.SemaphoreType.DMA,
    ],
    compiler_params=pltpu.CompilerParams(use_tc_tiling_on_sc=False),
)
def _gather_all(idx_hbm, table_hbm, out_hbm, idx_v, rows_v,
                isem0, isem1, gsem0, gsem1, osem0, osem1):
    wid = lax.axis_index("s") * 2 + lax.axis_index("c")
    base = wid * _PER_W
    isems = (isem0, isem1)
    gsems = (gsem0, gsem1)
    osems = (osem0, osem1)
    last_off = base + (_N_CHUNKS - 1) * _CHUNK

    def idx_copy(g, p):
        off = lax.min(base + g * _CHUNK, last_off)
        return pltpu.make_async_copy(
            idx_hbm.at[pl.ds(off, _CHUNK)], idx_v.at[p], isems[p])

    def gather_copy(p):
        return pltpu.make_async_copy(
            table_hbm.at[idx_v.at[p]], rows_v.at[p], gsems[p])

    def out_copy(g, p):
        off = base + g * _CHUNK
        return pltpu.make_async_copy(
            rows_v.at[p], out_hbm.at[pl.ds(off, _CHUNK)], osems[p])

    idx_copy(0, 0).start()
    idx_copy(1, 1).start()

    for p in (0, 1):
        idx_copy(p, p).wait()
        gather_copy(p).start()
        gather_copy(p).wait()
        out_copy(p, p).start()
        idx_copy(p + 2, p).start()

    def body(t, carry):
        for p in (0, 1):
            g = 2 * t + p
            out_copy(g - 2, p).wait()
            idx_copy(g, p).wait()
            gather_copy(p).start()
            gather_copy(p).wait()
            out_copy(g, p).start()
            idx_copy(g + 2, p).start()
        return carry

    lax.fori_loop(1, _N_CHUNKS // 2, body, 0)

    for p in (0, 1):
        out_copy(_N_CHUNKS - 2 + p, p).wait()
        idx_copy(_N_CHUNKS - 2 + p, p).wait()


def kernel(node_ids, embedding_weight):
    idx = node_ids.reshape(_TOTAL).astype(jnp.int32)
    out = _gather_all(idx, embedding_weight)
    return out.reshape(_BATCH, _HIST, _DIM)


# speedup vs baseline: 1.0762x; 1.0007x over previous
import functools

import jax
import jax.numpy as jnp
from jax import lax
from jax.experimental import pallas as pl
from jax.experimental.pallas import tpu as pltpu
from jax.experimental.pallas import tpu_sc as plsc

_BATCH = 16384
_HIST = 200
_DIM = 64
_TOTAL = _BATCH * _HIST
_NW = 32
_PER_W = _TOTAL // _NW
_CHUNK = 400
_NBUF = 4
_N_CHUNKS = _PER_W // _CHUNK

_mesh = plsc.VectorSubcoreMesh(core_axis_name="c", subcore_axis_name="s")


@functools.partial(
    pl.kernel,
    out_type=jax.ShapeDtypeStruct((_TOTAL, _DIM), jnp.float32),
    mesh=_mesh,
    scratch_types=[
        pltpu.VMEM((_NBUF, _CHUNK), jnp.int32),
        pltpu.VMEM((_NBUF, _CHUNK, _DIM), jnp.float32),
    ] + [pltpu.SemaphoreType.DMA] * (3 * _NBUF),
    compiler_params=pltpu.CompilerParams(use_tc_tiling_on_sc=False),
)
def _gather_all(idx_hbm, table_hbm, out_hbm, idx_v, rows_v, *sems):
    isems = sems[0:_NBUF]
    gsems = sems[_NBUF:2 * _NBUF]
    osems = sems[2 * _NBUF:3 * _NBUF]
    wid = lax.axis_index("s") * 2 + lax.axis_index("c")
    base = wid * _PER_W
    last_off = base + (_N_CHUNKS - 1) * _CHUNK

    def idx_copy(g, p):
        off = lax.min(base + g * _CHUNK, last_off)
        return pltpu.make_async_copy(
            idx_hbm.at[pl.ds(off, _CHUNK)], idx_v.at[p], isems[p])

    def gather_copy(p):
        return pltpu.make_async_copy(
            table_hbm.at[idx_v.at[p]], rows_v.at[p], gsems[p])

    def out_copy(g, p):
        off = base + g * _CHUNK
        return pltpu.make_async_copy(
            rows_v.at[p], out_hbm.at[pl.ds(off, _CHUNK)], osems[p])

    idx_copy(0, 0).start()
    idx_copy(1, 1).start()

    idx_copy(0, 0).wait()
    gather_copy(0).start()
    idx_copy(2, 2).start()
    idx_copy(1, 1).wait()
    gather_copy(1).start()
    idx_copy(3, 3).start()
    idx_copy(2, 2).wait()
    gather_copy(2).start()
    gather_copy(0).wait()
    out_copy(0, 0).start()
    idx_copy(4, 0).start()
    idx_copy(3, 3).wait()
    gather_copy(3).start()
    gather_copy(1).wait()
    out_copy(1, 1).start()
    idx_copy(5, 1).start()

    def body(t, carry):
        for p in (0, 1, 2, 3):
            g = 4 * t + p
            q = (p + 2) % 4
            out_copy(g - 4, p).wait()
            idx_copy(g, p).wait()
            gather_copy(p).start()
            gather_copy(q).wait()
            out_copy(g - 2, q).start()
            idx_copy(g + 2, q).start()
        return carry

    lax.fori_loop(1, _N_CHUNKS // 4, body, 0)

    g_last = _N_CHUNKS - 1
    gather_copy(2).wait()
    out_copy(g_last - 1, 2).start()
    gather_copy(3).wait()
    out_copy(g_last, 3).start()
    out_copy(g_last - 3, 0).wait()
    out_copy(g_last - 2, 1).wait()
    out_copy(g_last - 1, 2).wait()
    out_copy(g_last, 3).wait()
    idx_copy(_N_CHUNKS, 0).wait()
    idx_copy(_N_CHUNKS + 1, 1).wait()


def kernel(node_ids, embedding_weight):
    idx = node_ids.reshape(_TOTAL).astype(jnp.int32)
    out = _gather_all(idx, embedding_weight)
    return out.reshape(_BATCH, _HIST, _DIM)


# speedup vs baseline: 1.7757x; 1.6499x over previous
import functools

import jax
import jax.numpy as jnp
from jax import lax
from jax.experimental import pallas as pl
from jax.experimental.pallas import tpu as pltpu
from jax.experimental.pallas import tpu_sc as plsc

_BATCH = 16384
_HIST = 200
_DIM = 64
_TOTAL = _BATCH * _HIST
_NW = 32
_PER_W = _TOTAL // _NW
_CHUNK = 400
_NBUF = 4
_N_CHUNKS = _PER_W // _CHUNK

_mesh = plsc.VectorSubcoreMesh(core_axis_name="c", subcore_axis_name="s")


@functools.partial(
    pl.kernel,
    out_type=jax.ShapeDtypeStruct((_BATCH, _HIST, 2 * _DIM), jnp.float32),
    mesh=_mesh,
    scratch_types=[
        pltpu.VMEM((_NBUF, _CHUNK), jnp.int32),
        pltpu.VMEM((_NBUF, _CHUNK, _DIM), jnp.float32),
    ] + [pltpu.SemaphoreType.DMA] * (3 * _NBUF),
    compiler_params=pltpu.CompilerParams(use_tc_tiling_on_sc=False),
)
def _gather_all(idx_hbm, table_hbm, out_hbm, idx_v, rows_v, *sems):
    isems = sems[0:_NBUF]
    gsems = sems[_NBUF:2 * _NBUF]
    osems = sems[2 * _NBUF:3 * _NBUF]
    wid = lax.axis_index("s") * 2 + lax.axis_index("c")
    base = wid * _PER_W
    last_off = base + (_N_CHUNKS - 1) * _CHUNK

    def idx_copy(g, p):
        off = lax.min(base + g * _CHUNK, last_off)
        return pltpu.make_async_copy(
            idx_hbm.at[pl.ds(off, _CHUNK)], idx_v.at[p], isems[p])

    def gather_copy(p):
        return pltpu.make_async_copy(
            table_hbm.at[idx_v.at[p]], rows_v.at[p], gsems[p])

    def out_copies(g, p):
        b0 = (base + g * _CHUNK) // _HIST
        return [
            pltpu.make_async_copy(
                rows_v.at[p, pl.ds(r * _HIST, _HIST)],
                out_hbm.at[b0 + r, :, pl.ds(0, _DIM)],
                osems[p])
            for r in range(_CHUNK // _HIST)
        ]

    def out_start(g, p):
        for c in out_copies(g, p):
            c.start()

    def out_wait(g, p):
        for c in out_copies(g, p):
            c.wait()

    idx_copy(0, 0).start()
    idx_copy(1, 1).start()

    idx_copy(0, 0).wait()
    gather_copy(0).start()
    idx_copy(2, 2).start()
    idx_copy(1, 1).wait()
    gather_copy(1).start()
    idx_copy(3, 3).start()
    idx_copy(2, 2).wait()
    gather_copy(2).start()
    gather_copy(0).wait()
    out_start(0, 0)
    idx_copy(4, 0).start()
    idx_copy(3, 3).wait()
    gather_copy(3).start()
    gather_copy(1).wait()
    out_start(1, 1)
    idx_copy(5, 1).start()

    def body(t, carry):
        for p in (0, 1, 2, 3):
            g = 4 * t + p
            q = (p + 2) % 4
            out_wait(g - 4, p)
            idx_copy(g, p).wait()
            gather_copy(p).start()
            gather_copy(q).wait()
            out_start(g - 2, q)
            idx_copy(g + 2, q).start()
        return carry

    lax.fori_loop(1, _N_CHUNKS // 4, body, 0)

    g_last = _N_CHUNKS - 1
    gather_copy(2).wait()
    out_start(g_last - 1, 2)
    gather_copy(3).wait()
    out_start(g_last, 3)
    out_wait(g_last - 3, 0)
    out_wait(g_last - 2, 1)
    out_wait(g_last - 1, 2)
    out_wait(g_last, 3)
    idx_copy(_N_CHUNKS, 0).wait()
    idx_copy(_N_CHUNKS + 1, 1).wait()


def kernel(node_ids, embedding_weight):
    idx = node_ids.reshape(_TOTAL).astype(jnp.int32)
    out = _gather_all(idx, embedding_weight)
    return out[:, :, :_DIM]
